# Initial kernel scaffold; baseline (speedup 1.0000x reference)
import functools

import jax
import jax.numpy as jnp
from jax import lax
from jax.experimental import pallas as pl
from jax.experimental.pallas import tpu as pltpu
from jax.experimental.pallas import tpu_sc as plsc

N = 10000
E = 320000
D = 128
NPAD = 10240
NC = 2
NS = 16
NW = NC * NS
EPW = E // NW
CH = 80
NCHUNK = EPW // CH
GR = CH // 16
ROWS_PT = NPAD // NS
TAU = 1.0 / (128.0 ** 0.5)

_f32 = jnp.float32
_i32 = jnp.int32


def _proj_body(z_ref, wq_ref, bq_ref, wk_ref, bk_ref, wv_ref, bv_ref,
               q_ref, k_ref, v_ref):
    z = z_ref[...]
    q_ref[...] = jnp.dot(z, wq_ref[...], preferred_element_type=_f32) + bq_ref[...]
    k_ref[...] = jnp.dot(z, wk_ref[...], preferred_element_type=_f32) + bk_ref[...]
    v_ref[...] = jnp.dot(z, wv_ref[...], preferred_element_type=_f32) + bv_ref[...]


def _project(z, wqt, bq, wkt, bk, wvt, bv):
    blk = 1000
    grid = (N // blk,)
    zspec = pl.BlockSpec((blk, D), lambda i: (i, 0))
    wspec = pl.BlockSpec((D, D), lambda i: (0, 0))
    bspec = pl.BlockSpec((1, D), lambda i: (0, 0))
    ospec = pl.BlockSpec((blk, D), lambda i: (i, 0))
    out = pl.pallas_call(
        _proj_body,
        grid=grid,
        in_specs=[zspec, wspec, bspec, wspec, bspec, wspec, bspec],
        out_specs=[ospec, ospec, ospec],
        out_shape=[jax.ShapeDtypeStruct((N, D), _f32)] * 3,
    )(z, wqt, bq, wkt, bk, wvt, bv)
    return out


def _iota16():
    return lax.iota(_i32, 16)


def _edge_body(k_hbm, q_hbm, v_hbm, src_hbm, dst_hbm, zrow_hbm, zd_hbm,
               expe_hbm, hu_hbm, dp_hbm,
               src_v, dst_v, k_rows, q_rows, v_rows, expe_buf, dstage,
               dbuf, dsum, hu_sh, d_sh, sem0, sem1, sem2):
    c = lax.axis_index("c")
    s = lax.axis_index("s")
    wid = s * NC + c

    pltpu.sync_copy(zrow_hbm, hu_sh.at[pl.ds(s * ROWS_PT, ROWS_PT)])
    pltpu.sync_copy(zd_hbm, d_sh.at[pl.ds(s * ROWS_PT, ROWS_PT)])
    pltpu.sync_copy(zd_hbm.at[pl.ds(0, CH)], dstage)
    plsc.subcore_barrier()

    zcol = jnp.zeros((16,), _i32)

    def chunk_body(ch, carry):
        base = wid * EPW + ch * CH
        pltpu.sync_copy(src_hbm.at[pl.ds(base, CH)], src_v)
        pltpu.sync_copy(dst_hbm.at[pl.ds(base, CH)], dst_v)
        ck = pltpu.async_copy(k_hbm.at[src_v], k_rows, sem0)
        cq = pltpu.async_copy(q_hbm.at[dst_v], q_rows, sem1)
        cv = pltpu.async_copy(v_hbm.at[src_v], v_rows, sem2)
        ck.wait()
        cq.wait()
        cv.wait()
        for g in range(GR):
            rows = g * 16 + _iota16()

            def dot_body(j, acc):
                cj = jnp.full((16,), j, dtype=_i32)
                kj = plsc.load_gather(k_rows, [rows, cj])
                qj = plsc.load_gather(q_rows, [rows, cj])
                return acc + kj * qj

            acc = lax.fori_loop(0, D, dot_body, jnp.zeros((16,), _f32))
            ev = jnp.exp(acc * TAU)
            expe_buf[pl.ds(g * 16, 16)] = ev
            plsc.store_scatter(dstage, [rows, zcol], ev)

            def scale_body(j, carry2):
                cj = jnp.full((16,), j, dtype=_i32)
                vj = plsc.load_gather(v_rows, [rows, cj])
                plsc.store_scatter(v_rows, [rows, cj], vj * ev)
                return carry2

            lax.fori_loop(0, D, scale_body, 0)
        pltpu.sync_copy(v_rows, hu_sh.at[dst_v], add=True)
        pltpu.sync_copy(dstage, d_sh.at[dst_v], add=True)
        pltpu.sync_copy(expe_buf, expe_hbm.at[pl.ds(base, CH)])
        return carry

    lax.fori_loop(0, NCHUNK, chunk_body, 0)
    plsc.subcore_barrier()

    pltpu.sync_copy(hu_sh.at[pl.ds(s * ROWS_PT, ROWS_PT)],
                    hu_hbm.at[c, pl.ds(s * ROWS_PT, ROWS_PT)])
    pltpu.sync_copy(d_sh.at[pl.ds(s * ROWS_PT, ROWS_PT)], dbuf)

    def red_body(grp, carry):
        rows = grp * 16 + _iota16()
        acc = jnp.zeros((16,), _f32)
        for col in range(8):
            cj = jnp.full((16,), col, dtype=_i32)
            acc = acc + plsc.load_gather(dbuf, [rows, cj])
        dsum[pl.ds(grp * 16, 16)] = acc
        return carry

    lax.fori_loop(0, ROWS_PT // 16, red_body, 0)
    pltpu.sync_copy(dsum, dp_hbm.at[c, pl.ds(s * ROWS_PT, ROWS_PT)])


def _edge_phase(k, q, v, src, dst):
    zrow = jnp.zeros((ROWS_PT, D), _f32)
    zd = jnp.zeros((ROWS_PT, 8), _f32)
    mesh = plsc.VectorSubcoreMesh(core_axis_name="c", subcore_axis_name="s")
    fn = pl.kernel(
        _edge_body,
        out_type=[
            jax.ShapeDtypeStruct((E,), _f32),
            jax.ShapeDtypeStruct((NC, NPAD, D), _f32),
            jax.ShapeDtypeStruct((NC, NPAD), _f32),
        ],
        mesh=mesh,
        scratch_types=[
            pltpu.VMEM((CH,), _i32),
            pltpu.VMEM((CH,), _i32),
            pltpu.VMEM((CH, D), _f32),
            pltpu.VMEM((CH, D), _f32),
            pltpu.VMEM((CH, D), _f32),
            pltpu.VMEM((CH,), _f32),
            pltpu.VMEM((CH, 8), _f32),
            pltpu.VMEM((ROWS_PT, 8), _f32),
            pltpu.VMEM((ROWS_PT,), _f32),
            pltpu.VMEM_SHARED((NPAD, D), _f32),
            pltpu.VMEM_SHARED((NPAD, 8), _f32),
            pltpu.SemaphoreType.DMA,
            pltpu.SemaphoreType.DMA,
            pltpu.SemaphoreType.DMA,
        ],
    )
    return fn(k, q, v, src, dst, zrow, zd)


_NGROUPS = N // 16


def _final_body(hu_hbm, dp_hbm, expe_hbm, dst_hbm,
                h_hbm, alpha_hbm,
                d0, d1, dall, hu0, hu1, hout, rbuf, ev_buf, dste_buf, alpha_buf):
    c = lax.axis_index("c")
    s = lax.axis_index("s")
    wid = s * NC + c

    pltpu.sync_copy(dp_hbm.at[0], d0)
    pltpu.sync_copy(dp_hbm.at[1], d1)

    def add_body(i, carry):
        sl = pl.ds(i * 16, 16)
        dall[sl] = d0[sl] + d1[sl]
        return carry

    lax.fori_loop(0, NPAD // 16, add_body, 0)

    def h_body(t, carry):
        g = wid + t * NW

        @pl.when(g < _NGROUPS)
        def _():
            nb = g * 16
            pltpu.sync_copy(hu_hbm.at[0, pl.ds(nb, 16)], hu0)
            pltpu.sync_copy(hu_hbm.at[1, pl.ds(nb, 16)], hu1)
            dg = dall[pl.ds(nb, 16)]
            rbuf[...] = jnp.where(dg > 0.0, 1.0 / dg, 0.0)
            for r in range(16):
                sc = rbuf[r]
                for cc in range(D // 16):
                    sl = pl.ds(cc * 16, 16)
                    hout[r, sl] = (hu0[r, sl] + hu1[r, sl]) * sc
            pltpu.sync_copy(hout, h_hbm.at[pl.ds(nb, 16)])

        return carry

    lax.fori_loop(0, (_NGROUPS + NW - 1) // NW, h_body, 0)

    def a_body(ch, carry):
        base = wid * EPW + ch * CH
        pltpu.sync_copy(expe_hbm.at[pl.ds(base, CH)], ev_buf)
        pltpu.sync_copy(dst_hbm.at[pl.ds(base, CH)], dste_buf)
        for g in range(GR):
            sl = pl.ds(g * 16, 16)
            dv = plsc.load_gather(dall, [dste_buf[sl]])
            alpha_buf[sl] = ev_buf[sl] / dv
        pltpu.sync_copy(alpha_buf, alpha_hbm.at[pl.ds(base, CH)])
        return carry

    lax.fori_loop(0, NCHUNK, a_body, 0)


def _finalize(hu_p, dp, expe, dst):
    mesh = plsc.VectorSubcoreMesh(core_axis_name="c", subcore_axis_name="s")
    fn = pl.kernel(
        _final_body,
        out_type=[
            jax.ShapeDtypeStruct((N, D), _f32),
            jax.ShapeDtypeStruct((E,), _f32),
        ],
        mesh=mesh,
        scratch_types=[
            pltpu.VMEM((NPAD,), _f32),
            pltpu.VMEM((NPAD,), _f32),
            pltpu.VMEM((NPAD,), _f32),
            pltpu.VMEM((16, D), _f32),
            pltpu.VMEM((16, D), _f32),
            pltpu.VMEM((16, D), _f32),
            pltpu.VMEM((16,), _f32),
            pltpu.VMEM((CH,), _f32),
            pltpu.VMEM((CH,), _i32),
            pltpu.VMEM((CH,), _f32),
        ],
    )
    return fn(hu_p, dp, expe, dst)


def kernel(z, edge_index, Wq, bq, Wk, bk, Wv, bv):
    q, k, v = _project(z, Wq.T, bq.reshape(1, D), Wk.T, bk.reshape(1, D),
                       Wv.T, bv.reshape(1, D))
    src = edge_index[0]
    dst = edge_index[1]
    expe, hu_p, dp = _edge_phase(k, q, v, src, dst)
    h, alpha = _finalize(hu_p, dp, expe, dst)
    return h, alpha


# speedup vs baseline: 2.4313x; 2.4313x over previous
import functools

import jax
import jax.numpy as jnp
from jax import lax
from jax.experimental import pallas as pl
from jax.experimental.pallas import tpu as pltpu
from jax.experimental.pallas import tpu_sc as plsc

N = 10000
E = 320000
D = 128
NPAD = 10240
NC = 2
NS = 16
NW = NC * NS
EPW = E // NW
CH = 80
NCHUNK = EPW // CH
GR = CH // 16
ROWS_PT = NPAD // NS
TAU = 1.0 / (128.0 ** 0.5)

_f32 = jnp.float32
_i32 = jnp.int32


def _proj_body(z_ref, wq_ref, bq_ref, wk_ref, bk_ref, wv_ref, bv_ref,
               q_ref, k_ref, v_ref):
    z = z_ref[...]
    q_ref[...] = jnp.dot(z, wq_ref[...], preferred_element_type=_f32) + bq_ref[...]
    k_ref[...] = jnp.dot(z, wk_ref[...], preferred_element_type=_f32) + bk_ref[...]
    v_ref[...] = jnp.dot(z, wv_ref[...], preferred_element_type=_f32) + bv_ref[...]


def _project(z, wqt, bq, wkt, bk, wvt, bv):
    blk = 1000
    grid = (N // blk,)
    zspec = pl.BlockSpec((blk, D), lambda i: (i, 0))
    wspec = pl.BlockSpec((D, D), lambda i: (0, 0))
    bspec = pl.BlockSpec((1, D), lambda i: (0, 0))
    ospec = pl.BlockSpec((blk, D), lambda i: (i, 0))
    out = pl.pallas_call(
        _proj_body,
        grid=grid,
        in_specs=[zspec, wspec, bspec, wspec, bspec, wspec, bspec],
        out_specs=[ospec, ospec, ospec],
        out_shape=[jax.ShapeDtypeStruct((N, D), _f32)] * 3,
    )(z, wqt, bq, wkt, bk, wvt, bv)
    return out


def _iota16():
    return lax.iota(_i32, 16)


def _edge_body(k_hbm, q_hbm, v_hbm, src_hbm, dst_hbm, zrow_hbm, zd_hbm,
               expe_hbm, hu_hbm, dp_hbm,
               src_v, dst_v, k_rows, q_rows, v_rows, expe_buf,
               hu_sh, d_sh, sem0, sem1, sem2):
    c = lax.axis_index("c")
    s = lax.axis_index("s")
    wid = s * NC + c

    pltpu.sync_copy(zrow_hbm, hu_sh.at[pl.ds(s * ROWS_PT, ROWS_PT)])
    pltpu.sync_copy(zd_hbm, d_sh.at[pl.ds(s * ROWS_PT, ROWS_PT)])
    plsc.subcore_barrier()

    def chunk_body(ch, carry):
        base = wid * EPW + ch * CH
        pltpu.sync_copy(src_hbm.at[pl.ds(base, CH)], src_v)
        pltpu.sync_copy(dst_hbm.at[pl.ds(base, CH)], dst_v)
        ck = pltpu.async_copy(k_hbm.at[src_v], k_rows, sem0)
        cq = pltpu.async_copy(q_hbm.at[dst_v], q_rows, sem1)
        cv = pltpu.async_copy(v_hbm.at[src_v], v_rows, sem2)
        ck.wait()
        cq.wait()
        cv.wait()
        for g in range(GR):
            rows = g * 16 + _iota16()

            def dot_body(j, acc):
                cj = jnp.full((16,), j, dtype=_i32)
                kj = plsc.load_gather(k_rows, [rows, cj])
                qj = plsc.load_gather(q_rows, [rows, cj])
                return acc + kj * qj

            acc = lax.fori_loop(0, D, dot_body, jnp.zeros((16,), _f32))
            ev = jnp.exp(acc * TAU)
            expe_buf[pl.ds(g * 16, 16)] = ev

            def scale_body(j, carry2):
                cj = jnp.full((16,), j, dtype=_i32)
                vj = plsc.load_gather(v_rows, [rows, cj])
                plsc.store_scatter(v_rows, [rows, cj], vj * ev)
                return carry2

            lax.fori_loop(0, D, scale_body, 0)
        pltpu.sync_copy(v_rows, hu_sh.at[dst_v], add=True)
        pltpu.sync_copy(expe_buf, d_sh.at[dst_v], add=True)
        pltpu.sync_copy(expe_buf, expe_hbm.at[pl.ds(base, CH)])
        return carry

    lax.fori_loop(0, NCHUNK, chunk_body, 0)
    plsc.subcore_barrier()

    pltpu.sync_copy(hu_sh.at[pl.ds(s * ROWS_PT, ROWS_PT)],
                    hu_hbm.at[c, pl.ds(s * ROWS_PT, ROWS_PT)])
    pltpu.sync_copy(d_sh.at[pl.ds(s * ROWS_PT, ROWS_PT)],
                    dp_hbm.at[pl.ds(c * NPAD + s * ROWS_PT, ROWS_PT)])


def _edge_phase(k, q, v, src, dst):
    zrow = jnp.zeros((ROWS_PT, D), _f32)
    zd = jnp.zeros((ROWS_PT,), _f32)
    mesh = plsc.VectorSubcoreMesh(core_axis_name="c", subcore_axis_name="s")
    fn = pl.kernel(
        _edge_body,
        out_type=[
            jax.ShapeDtypeStruct((E,), _f32),
            jax.ShapeDtypeStruct((NC, NPAD, D), _f32),
            jax.ShapeDtypeStruct((NC * NPAD,), _f32),
        ],
        mesh=mesh,
        compiler_params=pltpu.CompilerParams(needs_layout_passes=False),
        scratch_types=[
            pltpu.VMEM((CH,), _i32),
            pltpu.VMEM((CH,), _i32),
            pltpu.VMEM((CH, D), _f32),
            pltpu.VMEM((CH, D), _f32),
            pltpu.VMEM((CH, D), _f32),
            pltpu.VMEM((CH,), _f32),
            pltpu.VMEM_SHARED((NPAD, D), _f32),
            pltpu.VMEM_SHARED((NPAD,), _f32),
            pltpu.SemaphoreType.DMA,
            pltpu.SemaphoreType.DMA,
            pltpu.SemaphoreType.DMA,
        ],
    )
    return fn(k, q, v, src, dst, zrow, zd)


_NGROUPS = N // 16


def _final_body(hu0_hbm, hu1_hbm, dp_hbm, expe_hbm, dst_hbm,
                h_hbm, alpha_hbm,
                dfbuf, dall, hu0, hu1, hout, ev_buf, dste_buf, alpha_buf):
    c = lax.axis_index("c")
    s = lax.axis_index("s")
    wid = s * NC + c

    pltpu.sync_copy(dp_hbm, dfbuf)

    def dred_body(i, carry):
        sl = pl.ds(i * 16, 16)
        dall[sl] = dfbuf[sl] + dfbuf[pl.ds(NPAD + i * 16, 16)]
        return carry

    lax.fori_loop(0, NPAD // 16, dred_body, 0)

    def h_body(t, carry):
        g = wid + t * NW
        nb = pl.multiple_of(g * 16, 16)
        pltpu.sync_copy(hu0_hbm.at[pl.ds(nb, 16)], hu0)
        pltpu.sync_copy(hu1_hbm.at[pl.ds(nb, 16)], hu1)
        dg = dall[pl.ds(nb, 16)]
        rcp = jnp.where(dg > 0.0, 1.0 / dg, 0.0)
        for r in range(16):
            sc = rcp[r]
            for cc in range(D // 16):
                sl = pl.ds(cc * 16, 16)
                hout[r, sl] = (hu0[r, sl] + hu1[r, sl]) * sc
        pltpu.sync_copy(hout, h_hbm.at[pl.ds(nb, 16)])
        return carry

    my_groups = (_NGROUPS - wid + NW - 1) // NW
    lax.fori_loop(0, my_groups, h_body, 0)

    def a_body(ch, carry):
        base = wid * EPW + ch * CH
        pltpu.sync_copy(expe_hbm.at[pl.ds(base, CH)], ev_buf)
        pltpu.sync_copy(dst_hbm.at[pl.ds(base, CH)], dste_buf)
        for g in range(GR):
            sl = pl.ds(g * 16, 16)
            dv = plsc.load_gather(dall, [dste_buf[sl]])
            alpha_buf[sl] = ev_buf[sl] / dv
        pltpu.sync_copy(alpha_buf, alpha_hbm.at[pl.ds(base, CH)])
        return carry

    lax.fori_loop(0, NCHUNK, a_body, 0)


def _finalize(hu_p, dp, expe, dst):
    mesh = plsc.VectorSubcoreMesh(core_axis_name="c", subcore_axis_name="s")
    fn = pl.kernel(
        _final_body,
        out_type=[
            jax.ShapeDtypeStruct((N, D), _f32),
            jax.ShapeDtypeStruct((E,), _f32),
        ],
        mesh=mesh,
        compiler_params=pltpu.CompilerParams(needs_layout_passes=False),
        scratch_types=[
            pltpu.VMEM((NC * NPAD,), _f32),
            pltpu.VMEM((NPAD,), _f32),
            pltpu.VMEM((16, D), _f32),
            pltpu.VMEM((16, D), _f32),
            pltpu.VMEM((16, D), _f32),
            pltpu.VMEM((CH,), _f32),
            pltpu.VMEM((CH,), _i32),
            pltpu.VMEM((CH,), _f32),
        ],
    )
    return fn(hu_p[0], hu_p[1], dp, expe, dst)


def kernel(z, edge_index, Wq, bq, Wk, bk, Wv, bv):
    q, k, v = _project(z, Wq.T, bq.reshape(1, D), Wk.T, bk.reshape(1, D),
                       Wv.T, bv.reshape(1, D))
    src = edge_index[0]
    dst = edge_index[1]
    expe, hu_p, dp = _edge_phase(k, q, v, src, dst)
    h, alpha = _finalize(hu_p, dp, expe, dst)
    return h, alpha
